# baseline (speedup 1.0000x reference)
import functools

import jax
import jax.numpy as jnp
from jax import lax
from jax.experimental import pallas as pl
from jax.experimental.pallas import tpu as pltpu
from jax.experimental.pallas import tpu_sc as plsc

DIM = 64
NC = 2
NS = 16
NW = NC * NS
SEQ_CHUNK = 16
N_CHUNKS = 8


@functools.cache
def _make_gather(B4, S):
    seq_per_w = B4 // NW
    chunk = SEQ_CHUNK * S
    assert seq_per_w == SEQ_CHUNK * N_CHUNKS
    mesh = plsc.VectorSubcoreMesh(core_axis_name="c", subcore_axis_name="s")

    @functools.partial(
        pl.kernel,
        mesh=mesh,
        compiler_params=pltpu.CompilerParams(use_tc_tiling_on_sc=False),
        out_type=jax.ShapeDtypeStruct((B4, S, DIM), jnp.float32),
        scratch_types=[
            pltpu.VMEM((N_CHUNKS, chunk), jnp.int32),
            pltpu.VMEM((chunk, DIM), jnp.float32),
            pltpu.VMEM((chunk, DIM), jnp.float32),
            pltpu.SemaphoreType.DMA,
            pltpu.SemaphoreType.DMA,
            pltpu.SemaphoreType.DMA,
        ],
    )
    def k(idx_hbm, table_hbm, out_hbm, idx_v, rows_a, rows_b, gsem_a, gsem_b, wsem):
        wid = lax.axis_index("s") * NC + lax.axis_index("c")
        base = wid * seq_per_w
        pltpu.sync_copy(idx_hbm.at[wid], idx_v)
        bufs = (rows_a, rows_b)
        gsems = (gsem_a, gsem_b)

        def start_gather(j, slot):
            return pltpu.async_copy(table_hbm.at[idx_v.at[j]], bufs[slot], gsems[slot])

        def start_writes(j, slot):
            b0 = base + j * SEQ_CHUNK
            return [
                pltpu.async_copy(bufs[slot].at[pl.ds(i * S, S)], out_hbm.at[b0 + i], wsem)
                for i in range(SEQ_CHUNK)
            ]

        gathers = [None] * N_CHUNKS
        writes = [None] * N_CHUNKS
        gathers[0] = start_gather(0, 0)
        for j in range(N_CHUNKS):
            slot = j % 2
            gathers[j].wait()
            if j + 1 < N_CHUNKS:
                if j >= 1:
                    for cp in writes[j - 1]:
                        cp.wait()
                gathers[j + 1] = start_gather(j + 1, 1 - slot)
            writes[j] = start_writes(j, slot)
        for cp in writes[N_CHUNKS - 2]:
            cp.wait()
        for cp in writes[N_CHUNKS - 1]:
            cp.wait()

    return k


def kernel(xs, weight):
    idx = xs.astype(jnp.int32).reshape(NW, N_CHUNKS, SEQ_CHUNK * xs.shape[1])
    return _make_gather(xs.shape[0], xs.shape[1])(idx, weight)
